# baseline (speedup 1.0000x reference)
import functools

import jax
import jax.numpy as jnp
from jax import lax
from jax.experimental import pallas as pl
from jax.experimental.pallas import tpu as pltpu
from jax.experimental.pallas import tpu_sc as plsc

_BLK = 128
_CH_BLK = 2
_NBUF = 4
_AHEAD = 3


@functools.lru_cache(maxsize=None)
def _make_gather(n_batch: int, n_points: int, d: int, k: int):
    rows_total = n_batch * n_points * k
    info = plsc.get_sparse_core_info()
    nw = info.num_cores * info.num_subcores
    rows_per_w = rows_total // nw
    n_chunks = rows_per_w // (_CH_BLK * _BLK)
    ch_rows = _CH_BLK * _BLK
    pts_per_ch = ch_rows // k
    lanes = info.num_lanes
    assert rows_total % (nw * ch_rows) == 0 and n_chunks % _NBUF == 0
    assert nw % n_batch == 0 and k == lanes

    mesh = plsc.VectorSubcoreMesh(core_axis_name="c", subcore_axis_name="s")

    @functools.partial(
        pl.kernel,
        mesh=mesh,
        compiler_params=pltpu.CompilerParams(use_tc_tiling_on_sc=False),
        out_type=jax.ShapeDtypeStruct((rows_total, d), jnp.float32),
        scratch_types=[
            pltpu.VMEM((rows_per_w // k, k), jnp.int32),
            [pltpu.VMEM((_CH_BLK, _BLK), jnp.int32)] * _NBUF,
            [pltpu.VMEM((ch_rows, d), jnp.float32)] * _NBUF,
            [pltpu.SemaphoreType.DMA] * _NBUF,
            [pltpu.SemaphoreType.DMA] * _NBUF,
        ],
    )
    def gather_kernel(x_hbm, idx_hbm, out_hbm,
                      slab, gidx, row_bufs, gsems, wsems):
        wid = lax.axis_index("s") * info.num_cores + lax.axis_index("c")
        bidx = wid // (nw // n_batch)
        p_base = (wid % (nw // n_batch)) * (rows_per_w // k)
        wrow0 = wid * rows_per_w
        table = x_hbm.at[bidx]

        pltpu.sync_copy(idx_hbm.at[bidx, pl.ds(p_base, rows_per_w // k)], slab)

        def repack(ci, b):
            for p in range(pts_per_ch):
                vec = slab[ci * pts_per_ch + p, :]
                gidx[b][p * k // _BLK, pl.ds((p * k) % _BLK, k)] = vec

        def start_gather(ci, b):
            for j in range(_CH_BLK):
                pltpu.async_copy(
                    table.at[gidx[b].at[j]],
                    row_bufs[b].at[pl.ds(j * _BLK, _BLK)],
                    gsems[b])

        def drain_gather(ci, b):
            for j in range(_CH_BLK):
                pltpu.make_async_copy(
                    table.at[gidx[b].at[j]],
                    row_bufs[b].at[pl.ds(j * _BLK, _BLK)],
                    gsems[b]).wait()

        def write_out(ci, b):
            pltpu.async_copy(
                row_bufs[b], out_hbm.at[pl.ds(wrow0 + ci * ch_rows, ch_rows)],
                wsems[b])

        def drain_write(ci, b):
            pltpu.make_async_copy(
                row_bufs[b], out_hbm.at[pl.ds(wrow0 + ci * ch_rows, ch_rows)],
                wsems[b]).wait()

        for ci in range(_AHEAD):
            repack(ci, ci % _NBUF)
            start_gather(ci, ci % _NBUF)

        def group(g, _):
            for b in range(_NBUF):
                ci = g * _NBUF + b
                drain_gather(ci, b)
                write_out(ci, b)
                nb = (b + _AHEAD) % _NBUF

                @pl.when(ci >= _NBUF - _AHEAD)
                def _():
                    drain_write(ci - (_NBUF - _AHEAD), nb)

                @pl.when(ci + _AHEAD < n_chunks)
                def _():
                    repack(ci + _AHEAD, nb)
                    start_gather(ci + _AHEAD, nb)
            return ()

        lax.fori_loop(0, n_chunks // _NBUF, group, (), unroll=False)

        for ci in range(n_chunks - (_NBUF - _AHEAD), n_chunks):
            drain_write(ci, ci % _NBUF)

    return gather_kernel


def kernel(x, refer_idx):
    b, n, d = x.shape
    k = refer_idx.shape[2]
    out = _make_gather(b, n, d, k)(x, refer_idx.astype(jnp.int32))
    return out.reshape(b, n, k * d)


# speedup vs baseline: 1.0666x; 1.0666x over previous
import functools

import jax
import jax.numpy as jnp
from jax import lax
from jax.experimental import pallas as pl
from jax.experimental.pallas import tpu as pltpu
from jax.experimental.pallas import tpu_sc as plsc

_BLK = 128
_CH_BLK = 2
_NBUF = 4
_AHEAD = 3


@functools.lru_cache(maxsize=None)
def _make_gather(n_batch: int, n_points: int, d: int, k: int):
    rows_total = n_batch * n_points * k
    info = plsc.get_sparse_core_info()
    nw = info.num_cores * info.num_subcores
    rows_per_w = rows_total // nw
    n_chunks = rows_per_w // (_CH_BLK * _BLK)
    ch_rows = _CH_BLK * _BLK
    pts_per_ch = ch_rows // k
    lanes = info.num_lanes
    assert rows_total % (nw * ch_rows) == 0 and n_chunks % _NBUF == 0
    assert nw % n_batch == 0 and k == lanes

    mesh = plsc.VectorSubcoreMesh(core_axis_name="c", subcore_axis_name="s")

    @functools.partial(
        pl.kernel,
        mesh=mesh,
        compiler_params=pltpu.CompilerParams(use_tc_tiling_on_sc=False),
        out_type=jax.ShapeDtypeStruct((rows_total, d), jnp.float32),
        scratch_types=[
            pltpu.VMEM((rows_per_w // k, k), jnp.int32),
            [pltpu.VMEM((_CH_BLK, _BLK), jnp.int32)] * _NBUF,
            [pltpu.VMEM((ch_rows, d), jnp.float32)] * _NBUF,
            [pltpu.SemaphoreType.DMA] * _NBUF,
            [pltpu.SemaphoreType.DMA] * _NBUF,
        ],
    )
    def gather_kernel(x_hbm, idx_hbm, out_hbm,
                      slab, gidx, row_bufs, gsems, wsems):
        wid = lax.axis_index("s") * info.num_cores + lax.axis_index("c")
        bidx = wid // (nw // n_batch)
        p_base = (wid % (nw // n_batch)) * (rows_per_w // k)
        wrow0 = wid * rows_per_w
        table = x_hbm.at[bidx]

        pltpu.sync_copy(idx_hbm.at[bidx, pl.ds(p_base, rows_per_w // k)], slab)

        def repack(ci, b):
            for p in range(pts_per_ch):
                vec = slab[ci * pts_per_ch + p, :]
                gidx[b][p * k // _BLK, pl.ds((p * k) % _BLK, k)] = vec

        def start_gather(ci, b):
            for j in range(_CH_BLK):
                pltpu.async_copy(
                    table.at[gidx[b].at[j]],
                    row_bufs[b].at[pl.ds(j * _BLK, _BLK)],
                    gsems[b])

        def drain_gather(ci, b):
            for j in range(_CH_BLK):
                pltpu.make_async_copy(
                    table.at[gidx[b].at[j]],
                    row_bufs[b].at[pl.ds(j * _BLK, _BLK)],
                    gsems[b]).wait()

        def write_out(ci, b):
            pltpu.async_copy(
                row_bufs[b], out_hbm.at[pl.ds(wrow0 + ci * ch_rows, ch_rows)],
                wsems[b])

        def drain_write(ci, b):
            pltpu.make_async_copy(
                row_bufs[b], out_hbm.at[pl.ds(wrow0 + ci * ch_rows, ch_rows)],
                wsems[b]).wait()

        for ci in range(_AHEAD):
            repack(ci, ci % _NBUF)
            start_gather(ci, ci % _NBUF)

        def group(g, _):
            for b in range(_NBUF):
                ci = g * _NBUF + b
                drain_gather(ci, b)
                write_out(ci, b)
                nb = (b + _AHEAD) % _NBUF

                @pl.when(ci >= _NBUF - _AHEAD)
                def _():
                    drain_write(ci - (_NBUF - _AHEAD), nb)

                @pl.when(ci + _AHEAD < n_chunks)
                def _():
                    repack(ci + _AHEAD, nb)
                    start_gather(ci + _AHEAD, nb)
            return ()

        lax.fori_loop(0, n_chunks // _NBUF, group, (), unroll=False)

        for ci in range(n_chunks - (_NBUF - _AHEAD), n_chunks):
            drain_write(ci, ci % _NBUF)

    return gather_kernel


@functools.lru_cache(maxsize=None)
def _make_relayout(n_batch: int, n_points: int, kd: int):
    bp = 512
    groups = kd // 128
    n_blk = n_points // bp

    def body(g_ref, o_ref):
        for q in range(groups):
            o_ref[0, :, q * 128:(q + 1) * 128] = g_ref[:, q, :]

    return pl.pallas_call(
        body,
        grid=(n_batch * n_blk,),
        in_specs=[pl.BlockSpec((bp, groups, 128), lambda i: (i, 0, 0))],
        out_specs=pl.BlockSpec((1, bp, kd), lambda i: (i // n_blk, i % n_blk, 0)),
        out_shape=jax.ShapeDtypeStruct((n_batch, n_points, kd), jnp.float32),
    )


def kernel(x, refer_idx):
    b, n, d = x.shape
    k = refer_idx.shape[2]
    rows = _make_gather(b, n, d, k)(x, refer_idx.astype(jnp.int32))
    rows3 = rows.reshape(b * n, (k * d) // 128, 128)
    return _make_relayout(b, n, k * d)(rows3)


# speedup vs baseline: 1.0952x; 1.0268x over previous
import functools

import jax
import jax.numpy as jnp
from jax import lax
from jax.experimental import pallas as pl
from jax.experimental.pallas import tpu as pltpu
from jax.experimental.pallas import tpu_sc as plsc

_BLK = 128
_CH_BLK = 2
_NBUF = 4
_AHEAD = 2
_PHASES = 4


@functools.lru_cache(maxsize=None)
def _make_gather(n_batch: int, n_points: int, d: int, k: int, phase: int):
    batches = n_batch // _PHASES
    rows_out = batches * n_points * k
    info = plsc.get_sparse_core_info()
    nw = info.num_cores * info.num_subcores
    rows_per_w = rows_out // nw
    n_chunks = rows_per_w // (_CH_BLK * _BLK)
    ch_rows = _CH_BLK * _BLK
    pts_per_ch = ch_rows // k
    lanes = info.num_lanes
    assert rows_out % (nw * ch_rows) == 0 and n_chunks % _NBUF == 0
    assert nw % batches == 0 and k == lanes
    w_per_b = nw // batches
    pts_per_w = rows_per_w // k

    mesh = plsc.VectorSubcoreMesh(core_axis_name="c", subcore_axis_name="s")

    @functools.partial(
        pl.kernel,
        mesh=mesh,
        compiler_params=pltpu.CompilerParams(use_tc_tiling_on_sc=False),
        out_type=jax.ShapeDtypeStruct((rows_out, d), jnp.float32),
        scratch_types=[
            pltpu.VMEM((pts_per_w, k), jnp.int32),
            [pltpu.VMEM((_CH_BLK, _BLK), jnp.int32)] * _NBUF,
            [pltpu.VMEM((ch_rows, d), jnp.float32)] * _NBUF,
            [pltpu.SemaphoreType.DMA] * _NBUF,
            [pltpu.SemaphoreType.DMA] * _NBUF,
        ],
    )
    def gather_kernel(x_hbm, idx_hbm, out_hbm,
                      slab, gidx, row_bufs, gsems, wsems):
        wid = lax.axis_index("s") * info.num_cores + lax.axis_index("c")
        bidx = phase * batches + wid // w_per_b
        p_base = (wid % w_per_b) * pts_per_w
        wrow0 = wid * rows_per_w
        table = x_hbm.at[bidx]

        pltpu.sync_copy(idx_hbm.at[bidx, pl.ds(p_base, pts_per_w)], slab)

        def repack(ci, b):
            for p in range(pts_per_ch):
                vec = slab[ci * pts_per_ch + p, :]
                gidx[b][p * k // _BLK, pl.ds((p * k) % _BLK, k)] = vec

        def start_gather(ci, b):
            for j in range(_CH_BLK):
                pltpu.async_copy(
                    table.at[gidx[b].at[j]],
                    row_bufs[b].at[pl.ds(j * _BLK, _BLK)],
                    gsems[b])

        def drain_gather(ci, b):
            for j in range(_CH_BLK):
                pltpu.make_async_copy(
                    table.at[gidx[b].at[j]],
                    row_bufs[b].at[pl.ds(j * _BLK, _BLK)],
                    gsems[b]).wait()

        def write_out(ci, b):
            pltpu.async_copy(
                row_bufs[b], out_hbm.at[pl.ds(wrow0 + ci * ch_rows, ch_rows)],
                wsems[b])

        def drain_write(ci, b):
            pltpu.make_async_copy(
                row_bufs[b], out_hbm.at[pl.ds(wrow0 + ci * ch_rows, ch_rows)],
                wsems[b]).wait()

        for ci in range(_AHEAD):
            repack(ci, ci % _NBUF)
            start_gather(ci, ci % _NBUF)

        def group(g, _):
            for b in range(_NBUF):
                ci = g * _NBUF + b
                drain_gather(ci, b)
                write_out(ci, b)
                nb = (b + _AHEAD) % _NBUF

                @pl.when(ci >= _NBUF - _AHEAD)
                def _():
                    drain_write(ci - (_NBUF - _AHEAD), nb)

                @pl.when(ci + _AHEAD < n_chunks)
                def _():
                    repack(ci + _AHEAD, nb)
                    start_gather(ci + _AHEAD, nb)
            return ()

        lax.fori_loop(0, n_chunks // _NBUF, group, (), unroll=False)

        for ci in range(n_chunks - (_NBUF - _AHEAD), n_chunks):
            drain_write(ci, ci % _NBUF)

    return gather_kernel


@functools.lru_cache(maxsize=None)
def _make_relayout(n_batch: int, n_points: int, kd: int, phase: int):
    bp = 512
    groups = kd // 128
    n_blk = n_points // bp
    batches = n_batch // _PHASES
    grid = batches * n_blk
    b0 = phase * batches

    out_shape = jax.ShapeDtypeStruct((n_batch, n_points, kd), jnp.float32)
    in_specs = [pl.BlockSpec((bp, groups, 128), lambda i: (i, 0, 0))]
    kwargs = {}
    if phase == 0:
        def body(g_ref, o_ref):
            for q in range(groups):
                o_ref[0, :, q * 128:(q + 1) * 128] = g_ref[:, q, :]
    else:
        def body(g_ref, prev_ref, o_ref):
            del prev_ref
            for q in range(groups):
                o_ref[0, :, q * 128:(q + 1) * 128] = g_ref[:, q, :]
        in_specs.append(pl.BlockSpec(memory_space=pl.ANY))
        kwargs["input_output_aliases"] = {1: 0}

    return pl.pallas_call(
        body,
        grid=(grid,),
        in_specs=in_specs,
        out_specs=pl.BlockSpec((1, bp, kd),
                               lambda i: (b0 + i // n_blk, i % n_blk, 0)),
        out_shape=out_shape,
        **kwargs,
    )


def kernel(x, refer_idx):
    b, n, d = x.shape
    k = refer_idx.shape[2]
    idx = refer_idx.astype(jnp.int32)
    out = None
    for phase in range(_PHASES):
        rows = _make_gather(b, n, d, k, phase)(x, idx)
        rows3 = rows.reshape((b // _PHASES) * n, (k * d) // 128, 128)
        if phase == 0:
            out = _make_relayout(b, n, k * d, phase)(rows3)
        else:
            out = _make_relayout(b, n, k * d, phase)(rows3, out)
    return out


# speedup vs baseline: 1.1480x; 1.0482x over previous
import functools

import jax
import jax.numpy as jnp
from jax import lax
from jax.experimental import pallas as pl
from jax.experimental.pallas import tpu as pltpu
from jax.experimental.pallas import tpu_sc as plsc

_BLK = 128
_CH_BLK = 2
_NBUF = 4
_AHEAD = 2
_PHASES = 2


@functools.lru_cache(maxsize=None)
def _make_gather(n_batch: int, n_points: int, d: int, k: int, phase: int):
    batches = n_batch // _PHASES
    rows_out = batches * n_points * k
    info = plsc.get_sparse_core_info()
    nw = info.num_cores * info.num_subcores
    rows_per_w = rows_out // nw
    n_chunks = rows_per_w // (_CH_BLK * _BLK)
    ch_rows = _CH_BLK * _BLK
    pts_per_ch = ch_rows // k
    lanes = info.num_lanes
    assert rows_out % (nw * ch_rows) == 0 and n_chunks % _NBUF == 0
    assert nw % batches == 0 and k == lanes
    w_per_b = nw // batches
    pts_per_w = rows_per_w // k

    mesh = plsc.VectorSubcoreMesh(core_axis_name="c", subcore_axis_name="s")

    @functools.partial(
        pl.kernel,
        mesh=mesh,
        compiler_params=pltpu.CompilerParams(use_tc_tiling_on_sc=False),
        out_type=jax.ShapeDtypeStruct((rows_out, d), jnp.float32),
        scratch_types=[
            pltpu.VMEM((pts_per_w, k), jnp.int32),
            [pltpu.VMEM((_CH_BLK, _BLK), jnp.int32)] * _NBUF,
            [pltpu.VMEM((ch_rows, d), jnp.float32)] * _NBUF,
            [pltpu.SemaphoreType.DMA] * _NBUF,
            [pltpu.SemaphoreType.DMA] * _NBUF,
        ],
    )
    def gather_kernel(x_hbm, idx_hbm, out_hbm,
                      slab, gidx, row_bufs, gsems, wsems):
        wid = lax.axis_index("s") * info.num_cores + lax.axis_index("c")
        bidx = phase * batches + wid // w_per_b
        p_base = (wid % w_per_b) * pts_per_w
        wrow0 = wid * rows_per_w
        table = x_hbm.at[bidx]

        pltpu.sync_copy(idx_hbm.at[bidx, pl.ds(p_base, pts_per_w)], slab)

        def repack(ci, b):
            for p in range(pts_per_ch):
                vec = slab[ci * pts_per_ch + p, :]
                gidx[b][p * k // _BLK, pl.ds((p * k) % _BLK, k)] = vec

        def start_gather(ci, b):
            for j in range(_CH_BLK):
                pltpu.async_copy(
                    table.at[gidx[b].at[j]],
                    row_bufs[b].at[pl.ds(j * _BLK, _BLK)],
                    gsems[b])

        def drain_gather(ci, b):
            for j in range(_CH_BLK):
                pltpu.make_async_copy(
                    table.at[gidx[b].at[j]],
                    row_bufs[b].at[pl.ds(j * _BLK, _BLK)],
                    gsems[b]).wait()

        def write_out(ci, b):
            pltpu.async_copy(
                row_bufs[b], out_hbm.at[pl.ds(wrow0 + ci * ch_rows, ch_rows)],
                wsems[b])

        def drain_write(ci, b):
            pltpu.make_async_copy(
                row_bufs[b], out_hbm.at[pl.ds(wrow0 + ci * ch_rows, ch_rows)],
                wsems[b]).wait()

        for ci in range(_AHEAD):
            repack(ci, ci % _NBUF)
            start_gather(ci, ci % _NBUF)

        def group(g, _):
            for b in range(_NBUF):
                ci = g * _NBUF + b
                drain_gather(ci, b)
                write_out(ci, b)
                nb = (b + _AHEAD) % _NBUF

                @pl.when(ci >= _NBUF - _AHEAD)
                def _():
                    drain_write(ci - (_NBUF - _AHEAD), nb)

                @pl.when(ci + _AHEAD < n_chunks)
                def _():
                    repack(ci + _AHEAD, nb)
                    start_gather(ci + _AHEAD, nb)
            return ()

        lax.fori_loop(0, n_chunks // _NBUF, group, (), unroll=False)

        for ci in range(n_chunks - (_NBUF - _AHEAD), n_chunks):
            drain_write(ci, ci % _NBUF)

    return gather_kernel


@functools.lru_cache(maxsize=None)
def _make_relayout(n_batch: int, n_points: int, kd: int, phase: int):
    bp = 1024
    groups = kd // 128
    n_blk = n_points // bp
    batches = n_batch // _PHASES
    grid = batches * n_blk
    b0 = phase * batches

    out_shape = jax.ShapeDtypeStruct((n_batch, n_points, kd), jnp.float32)
    in_specs = [pl.BlockSpec((bp, groups, 128), lambda i: (i, 0, 0))]
    kwargs = {}
    if phase == 0:
        def body(g_ref, o_ref):
            for q in range(groups):
                o_ref[0, :, q * 128:(q + 1) * 128] = g_ref[:, q, :]
    else:
        def body(g_ref, prev_ref, o_ref):
            del prev_ref
            for q in range(groups):
                o_ref[0, :, q * 128:(q + 1) * 128] = g_ref[:, q, :]
        in_specs.append(pl.BlockSpec(memory_space=pl.ANY))
        kwargs["input_output_aliases"] = {1: 0}

    return pl.pallas_call(
        body,
        grid=(grid,),
        in_specs=in_specs,
        out_specs=pl.BlockSpec((1, bp, kd),
                               lambda i: (b0 + i // n_blk, i % n_blk, 0)),
        out_shape=out_shape,
        **kwargs,
    )


def kernel(x, refer_idx):
    b, n, d = x.shape
    k = refer_idx.shape[2]
    idx = refer_idx.astype(jnp.int32)
    out = None
    for phase in range(_PHASES):
        rows = _make_gather(b, n, d, k, phase)(x, idx)
        rows3 = rows.reshape((b // _PHASES) * n, (k * d) // 128, 128)
        if phase == 0:
            out = _make_relayout(b, n, k * d, phase)(rows3)
        else:
            out = _make_relayout(b, n, k * d, phase)(rows3, out)
    return out
